# baseline (speedup 1.0000x reference)
import functools
import math

import jax
import jax.numpy as jnp
from jax import lax
from jax.experimental import pallas as pl
from jax.experimental.pallas import tpu as pltpu, tpu_sc as plsc

_NC = 2
_NS = 16
_LANES = 16


def _make_embed_kernel(B, S, D, N):
    NW = _NC * _NS
    SPW = S // NW
    CH = 32
    CPB = SPW // CH
    NCH = CPB * B
    NBUF = 3
    scale = math.sqrt(D)

    mesh = plsc.VectorSubcoreMesh(
        core_axis_name="c", subcore_axis_name="s",
        num_cores=_NC, num_subcores=_NS)

    @functools.partial(
        pl.kernel,
        out_type=jax.ShapeDtypeStruct((N, D), jnp.float32),
        mesh=mesh,
        scratch_types=[
            pltpu.VMEM((NCH, CH), jnp.int32),
            pltpu.VMEM((CH * D // 2,), jnp.int32),
            pltpu.VMEM((CH, D), jnp.float32),
            pltpu.VMEM((CH, D), jnp.float32),
            pltpu.VMEM((CH, D), jnp.float32),
            pltpu.SemaphoreType.DMA,
            pltpu.SemaphoreType.DMA,
            pltpu.SemaphoreType.DMA,
            pltpu.SemaphoreType.DMA,
            pltpu.SemaphoreType.DMA,
            pltpu.SemaphoreType.DMA,
        ],
    )
    def embed(x_hbm, table_hbm, pos_hbm, out_hbm,
              idx_v, pos_v, rows0, rows1, rows2, g0, g1, g2, w0, w1, w2):
        wid = lax.axis_index("s") * _NC + lax.axis_index("c")
        s_base = wid * SPW
        rows = (rows0, rows1, rows2)
        gsem = (g0, g1, g2)
        wsem = (w0, w1, w2)

        pltpu.sync_copy(x_hbm.at[wid], idx_v)
        gh = {0: pltpu.async_copy(table_hbm.at[idx_v.at[0]], rows0, g0)}
        wh = {}
        for t in range(NCH):
            b = t % B
            c = t // B
            buf = t % NBUF
            if t + 1 < NCH:
                nbuf = (t + 1) % NBUF
                if t + 1 >= NBUF:
                    wh[t + 1 - NBUF].wait()
                gh[t + 1] = pltpu.async_copy(
                    table_hbm.at[idx_v.at[t + 1]], rows[nbuf], gsem[nbuf])
            if b == 0:
                pltpu.sync_copy(
                    pos_hbm.at[pl.ds((s_base + c * CH) * (D // 2), CH * D // 2)],
                    pos_v)
            gh[t].wait()
            r_buf = rows[buf]

            def row_body(r, carry, r_buf=r_buf):
                for j in range(D // (2 * _LANES)):
                    w = pos_v[pl.ds(r * (D // 2) + j * _LANES, _LANES)]
                    pa = lax.bitcast_convert_type(
                        lax.shift_left(w, 16), jnp.float32)
                    pb = lax.bitcast_convert_type(
                        lax.bitwise_and(w, -65536), jnp.float32)
                    sla = pl.ds(2 * j * _LANES, _LANES)
                    slb = pl.ds((2 * j + 1) * _LANES, _LANES)
                    r_buf[r, sla] = r_buf[r, sla] * scale + pa
                    r_buf[r, slb] = r_buf[r, slb] * scale + pb
                return carry

            lax.fori_loop(0, CH, row_body, 0)
            wh[t] = pltpu.async_copy(
                r_buf, out_hbm.at[pl.ds(b * S + s_base + c * CH, CH)], wsem[buf])
        for t in range(max(0, NCH - NBUF), NCH):
            wh[t].wait()

    return embed


def kernel(x, token_table, pos_table):
    B, S = x.shape
    V, D = token_table.shape
    N = B * S
    NW = _NC * _NS
    SPW = S // NW
    CH = 32
    CPB = SPW // CH
    xt = (x.astype(jnp.int32)
          .reshape(B, NW, CPB, CH)
          .transpose(1, 2, 0, 3)
          .reshape(NW, CPB * B, CH))
    pos_words = lax.bitcast_convert_type(
        pos_table.astype(jnp.bfloat16)
        .reshape(S, D // 32, 2, 16)
        .transpose(0, 1, 3, 2)
        .reshape(S * D // 2, 2),
        jnp.int32)
    embed = _make_embed_kernel(B, S, D, N)
    out = embed(xt, token_table, pos_words)
    return out.reshape(B, S, D)


# speedup vs baseline: 2.0720x; 2.0720x over previous
import functools
import math

import jax
import jax.numpy as jnp
from jax import lax
from jax.experimental import pallas as pl
from jax.experimental.pallas import tpu as pltpu, tpu_sc as plsc

_NC = 2
_NS = 16
_LANES = 16


def _make_embed_kernel(B, S, D, N):
    NW = _NC * _NS
    SPW = S // NW
    CH = 16
    CPB = SPW // CH
    NCH = CPB * B
    NBUF = 4
    scale = math.sqrt(D)

    mesh = plsc.VectorSubcoreMesh(
        core_axis_name="c", subcore_axis_name="s",
        num_cores=_NC, num_subcores=_NS)

    rows_types = [pltpu.VMEM((CH, D), jnp.float32) for _ in range(NBUF)]
    sem_types = [pltpu.SemaphoreType.DMA for _ in range(2 * NBUF)]

    @functools.partial(
        pl.kernel,
        out_type=jax.ShapeDtypeStruct((N, D), jnp.float32),
        mesh=mesh,
        scratch_types=[
            pltpu.VMEM((NCH, CH), jnp.int32),
            pltpu.VMEM((CH, D), jnp.float32),
        ] + rows_types + sem_types,
    )
    def embed(x_hbm, table_hbm, pos_hbm, out_hbm, idx_v, pos_v, *bufs_and_sems):
        rows = bufs_and_sems[:NBUF]
        gsem = bufs_and_sems[NBUF:2 * NBUF]
        wsem = bufs_and_sems[2 * NBUF:]
        wid = lax.axis_index("s") * _NC + lax.axis_index("c")
        s_base = wid * SPW

        pltpu.sync_copy(x_hbm.at[wid], idx_v)
        gh = {}
        wh = {}
        for t in range(min(2, NCH)):
            gh[t] = pltpu.async_copy(
                table_hbm.at[idx_v.at[t]], rows[t % NBUF], gsem[t % NBUF])
        for t in range(NCH):
            b = t % B
            c = t // B
            buf = t % NBUF
            if t + 2 < NCH:
                nbuf = (t + 2) % NBUF
                if t >= 2:
                    wh[t - 2].wait()
                gh[t + 2] = pltpu.async_copy(
                    table_hbm.at[idx_v.at[t + 2]], rows[nbuf], gsem[nbuf])
            if b == 0:
                pltpu.sync_copy(pos_hbm.at[pl.ds(s_base + c * CH, CH)], pos_v)
            gh[t].wait()
            r_buf = rows[buf]

            def row_body(r, carry, r_buf=r_buf):
                for j in range(D // _LANES):
                    sl = pl.ds(j * _LANES, _LANES)
                    r_buf[r, sl] = r_buf[r, sl] * scale + pos_v[r, sl]
                return carry

            lax.fori_loop(0, CH, row_body, 0)
            wh[t] = pltpu.async_copy(
                r_buf, out_hbm.at[pl.ds(b * S + s_base + c * CH, CH)], wsem[buf])
        for t in range(max(0, NCH - NBUF), NCH):
            wh[t].wait()

    return embed


def kernel(x, token_table, pos_table):
    B, S = x.shape
    V, D = token_table.shape
    N = B * S
    NW = _NC * _NS
    SPW = S // NW
    CH = 16
    CPB = SPW // CH
    xt = (x.astype(jnp.int32)
          .reshape(B, NW, CPB, CH)
          .transpose(1, 2, 0, 3)
          .reshape(NW, CPB * B, CH))
    embed = _make_embed_kernel(B, S, D, N)
    out = embed(xt, token_table, pos_table)
    return out.reshape(B, S, D)


# speedup vs baseline: 2.2885x; 1.1045x over previous
import functools
import math

import jax
import jax.numpy as jnp
from jax import lax
from jax.experimental import pallas as pl
from jax.experimental.pallas import tpu as pltpu, tpu_sc as plsc

_NC = 2
_NS = 16
_LANES = 16


def _make_embed_kernel(B, S, D, N):
    NW = _NC * _NS
    SPW = S // NW
    CH = 16
    CPB = SPW // CH
    NCH = CPB * B
    NBUF = 4
    scale = math.sqrt(D)

    mesh = plsc.VectorSubcoreMesh(
        core_axis_name="c", subcore_axis_name="s",
        num_cores=_NC, num_subcores=_NS)

    rows_types = [pltpu.VMEM((CH, D), jnp.float32) for _ in range(NBUF)]
    pos_types = [pltpu.VMEM((CH, D), jnp.float32) for _ in range(2)]
    sem_types = [pltpu.SemaphoreType.DMA for _ in range(3 * NBUF + 3)]

    @functools.partial(
        pl.kernel,
        out_type=jax.ShapeDtypeStruct((N, D), jnp.float32),
        mesh=mesh,
        scratch_types=[pltpu.VMEM((NCH, CH), jnp.int32)]
        + pos_types + rows_types + sem_types,
    )
    def embed(x_hbm, table_hbm, pos_hbm, out_hbm, idx_v, *refs):
        pos = refs[:2]
        rows = refs[2:2 + NBUF]
        gsem = refs[2 + NBUF:2 + 2 * NBUF]
        wsem = refs[2 + 2 * NBUF:2 + 3 * NBUF]
        psem = refs[2 + 3 * NBUF:4 + 3 * NBUF]
        isem = refs[4 + 3 * NBUF]
        wid = lax.axis_index("s") * _NC + lax.axis_index("c")
        s_base = wid * SPW

        ih = {}
        for t in range(NCH):
            b = t % B
            c = t // B
            ih[t] = pltpu.async_copy(
                x_hbm.at[b, pl.ds(s_base + c * CH, CH)], idx_v.at[t], isem)
        ph = {0: pltpu.async_copy(
            pos_hbm.at[pl.ds(s_base, CH)], pos[0], psem[0])}
        gh = {}
        wh = {}
        for t in range(min(2, NCH)):
            ih[t].wait()
            gh[t] = pltpu.async_copy(
                table_hbm.at[idx_v.at[t]], rows[t % NBUF], gsem[t % NBUF])
        for t in range(NCH):
            b = t % B
            c = t // B
            buf = t % NBUF
            if t + 2 < NCH:
                nbuf = (t + 2) % NBUF
                if t >= 2:
                    wh[t - 2].wait()
                ih[t + 2].wait()
                gh[t + 2] = pltpu.async_copy(
                    table_hbm.at[idx_v.at[t + 2]], rows[nbuf], gsem[nbuf])
            if b == 0:
                if c + 1 < CPB:
                    ph[c + 1] = pltpu.async_copy(
                        pos_hbm.at[pl.ds(s_base + (c + 1) * CH, CH)],
                        pos[(c + 1) % 2], psem[(c + 1) % 2])
                ph[c].wait()
            gh[t].wait()
            r_buf = rows[buf]
            p_buf = pos[c % 2]

            def row_body(r, carry, r_buf=r_buf, p_buf=p_buf):
                for j in range(D // _LANES):
                    sl = pl.ds(j * _LANES, _LANES)
                    r_buf[r, sl] = r_buf[r, sl] * scale + p_buf[r, sl]
                return carry

            lax.fori_loop(0, CH, row_body, 0)
            wh[t] = pltpu.async_copy(
                r_buf, out_hbm.at[pl.ds(b * S + s_base + c * CH, CH)],
                wsem[buf])
        for t in range(max(0, NCH - NBUF), NCH):
            wh[t].wait()

    return embed


def kernel(x, token_table, pos_table):
    B, S = x.shape
    V, D = token_table.shape
    N = B * S
    embed = _make_embed_kernel(B, S, D, N)
    out = embed(x.astype(jnp.int32), token_table, pos_table)
    return out.reshape(B, S, D)


# speedup vs baseline: 2.3286x; 1.0175x over previous
import functools
import math

import jax
import jax.numpy as jnp
from jax import lax
from jax.experimental import pallas as pl
from jax.experimental.pallas import tpu as pltpu, tpu_sc as plsc

_NC = 2
_NS = 16
_LANES = 16


def _make_embed_kernel(B, S, D, N):
    NW = _NC * _NS
    SPW = S // NW
    CH = 8
    NG = SPW // CH
    NSET = 3
    NCHUNK = NG * B
    scale = math.sqrt(D)

    mesh = plsc.VectorSubcoreMesh(
        core_axis_name="c", subcore_axis_name="s",
        num_cores=_NC, num_subcores=_NS)

    rows_types = [pltpu.VMEM((CH, D), jnp.float32) for _ in range(NSET * B)]
    pos_types = [pltpu.VMEM((CH, D), jnp.float32) for _ in range(2)]
    sem_types = [pltpu.SemaphoreType.DMA for _ in range(2 * NSET + 3)]

    @functools.partial(
        pl.kernel,
        out_type=jax.ShapeDtypeStruct((N, D), jnp.float32),
        mesh=mesh,
        scratch_types=[pltpu.VMEM((NCHUNK, CH), jnp.int32)]
        + pos_types + rows_types + sem_types,
    )
    def embed(x_hbm, table_hbm, pos_hbm, out_hbm, idx_v, *refs):
        pos = refs[:2]
        rows = refs[2:2 + NSET * B]
        gsem = refs[2 + NSET * B:2 + NSET * B + NSET]
        wsem = refs[2 + NSET * B + NSET:2 + NSET * B + 2 * NSET]
        psem = refs[2 + NSET * B + 2 * NSET:4 + NSET * B + 2 * NSET]
        isem = refs[4 + NSET * B + 2 * NSET]
        wid = lax.axis_index("s") * _NC + lax.axis_index("c")
        s_base = wid * SPW

        ih = {}
        for t in range(NCHUNK):
            b = t % B
            g = t // B
            ih[t] = pltpu.async_copy(
                x_hbm.at[b, pl.ds(s_base + g * CH, CH)], idx_v.at[t], isem)
        ph = {0: pltpu.async_copy(
            pos_hbm.at[pl.ds(s_base, CH)], pos[0], psem[0])}

        def gather_group(g):
            st = g % NSET
            hs = []
            for b in range(B):
                ih[g * B + b].wait()
                hs.append(pltpu.async_copy(
                    table_hbm.at[idx_v.at[g * B + b]],
                    rows[st * B + b], gsem[st]))
            return hs

        gh = {0: gather_group(0)}
        wh = {}
        for g in range(NG):
            st = g % NSET
            if g + 1 < NG:
                if g >= 2:
                    for h in wh[g - 2]:
                        h.wait()
                gh[g + 1] = gather_group(g + 1)
                if g + 1 < NG:
                    ph[g + 1] = pltpu.async_copy(
                        pos_hbm.at[pl.ds(s_base + (g + 1) * CH, CH)],
                        pos[(g + 1) % 2], psem[(g + 1) % 2])
            ph[g].wait()
            for h in gh[g]:
                h.wait()
            p_buf = pos[g % 2]
            bufs = rows[st * B:st * B + B]

            def row_body(k, carry, bufs=bufs, p_buf=p_buf):
                r = k // 2
                half = k % 2
                for j in range(D // (2 * _LANES)):
                    sl = pl.ds(half * (D // 2) + j * _LANES, _LANES)
                    pv = p_buf[r, sl]
                    for bi in range(B):
                        bufs[bi][r, sl] = bufs[bi][r, sl] * scale + pv
                return carry

            lax.fori_loop(0, 2 * CH, row_body, 0)
            whl = []
            for b in range(B):
                whl.append(pltpu.async_copy(
                    bufs[b], out_hbm.at[pl.ds(b * S + s_base + g * CH, CH)],
                    wsem[st]))
            wh[g] = whl
        for g in range(max(0, NG - 2), NG):
            for h in wh[g]:
                h.wait()

    return embed


def kernel(x, token_table, pos_table):
    B, S = x.shape
    V, D = token_table.shape
    N = B * S
    embed = _make_embed_kernel(B, S, D, N)
    out = embed(x.astype(jnp.int32), token_table, pos_table)
    return out.reshape(B, S, D)


# speedup vs baseline: 2.3473x; 1.0080x over previous
import functools
import math

import jax
import jax.numpy as jnp
from jax import lax
from jax.experimental import pallas as pl
from jax.experimental.pallas import tpu as pltpu, tpu_sc as plsc

_NC = 2
_NS = 16
_LANES = 16


def _make_embed_kernel(B, S, D, N):
    NW = _NC * _NS
    SPW = S // NW
    CH = 8
    NG = SPW // CH
    NSET = 3
    NCHUNK = NG * B
    scale = math.sqrt(D)

    mesh = plsc.VectorSubcoreMesh(
        core_axis_name="c", subcore_axis_name="s",
        num_cores=_NC, num_subcores=_NS)

    rows_types = [pltpu.VMEM((CH, D), jnp.float32) for _ in range(NSET * B)]
    pos_types = [pltpu.VMEM((CH, D), jnp.float32) for _ in range(2)]
    sem_types = [pltpu.SemaphoreType.DMA for _ in range(2 * NSET + 3)]

    @functools.partial(
        pl.kernel,
        out_type=jax.ShapeDtypeStruct((N, D), jnp.float32),
        mesh=mesh,
        scratch_types=[pltpu.VMEM((NCHUNK, CH), jnp.int32)]
        + pos_types + rows_types + sem_types,
    )
    def embed(x_hbm, table_hbm, pos_hbm, out_hbm, idx_v, *refs):
        pos = refs[:2]
        rows = refs[2:2 + NSET * B]
        gsem = refs[2 + NSET * B:2 + NSET * B + NSET]
        wsem = refs[2 + NSET * B + NSET:2 + NSET * B + 2 * NSET]
        psem = refs[2 + NSET * B + 2 * NSET:4 + NSET * B + 2 * NSET]
        isem = refs[4 + NSET * B + 2 * NSET]
        wid = lax.axis_index("s") * _NC + lax.axis_index("c")
        s_base = wid * SPW

        ih = {}
        for t in range(NCHUNK):
            b = t % B
            g = t // B
            ih[t] = pltpu.async_copy(
                x_hbm.at[b, pl.ds(s_base + g * CH, CH)], idx_v.at[t], isem)
        ph = {0: pltpu.async_copy(
            pos_hbm.at[pl.ds(s_base, CH)], pos[0], psem[0])}

        def gather_group(g):
            st = g % NSET
            hs = []
            for b in range(B):
                ih[g * B + b].wait()
                hs.append(pltpu.async_copy(
                    table_hbm.at[idx_v.at[g * B + b]],
                    rows[st * B + b], gsem[st]))
            return hs

        gh = {0: gather_group(0)}
        wh = {}
        for g in range(NG):
            st = g % NSET
            if g + 1 < NG:
                ph[g + 1] = pltpu.async_copy(
                    pos_hbm.at[pl.ds(s_base + (g + 1) * CH, CH)],
                    pos[(g + 1) % 2], psem[(g + 1) % 2])
                if g >= 2:
                    for h in wh[g - 2]:
                        h.wait()
                gh[g + 1] = gather_group(g + 1)
            ph[g].wait()
            for h in gh[g]:
                h.wait()
            p_buf = pos[g % 2]
            bufs = rows[st * B:st * B + B]

            def row_body(k, carry, bufs=bufs, p_buf=p_buf):
                r = k // 2
                half = k % 2
                for j in range(D // (2 * _LANES)):
                    sl = pl.ds(half * (D // 2) + j * _LANES, _LANES)
                    pv = p_buf[r, sl]
                    for bi in range(B):
                        bufs[bi][r, sl] = bufs[bi][r, sl] * scale + pv
                return carry

            lax.fori_loop(0, 2 * CH, row_body, 0)
            whl = []
            for b in range(B):
                whl.append(pltpu.async_copy(
                    bufs[b], out_hbm.at[pl.ds(b * S + s_base + g * CH, CH)],
                    wsem[st]))
            wh[g] = whl
        for g in range(max(0, NG - 2), NG):
            for h in wh[g]:
                h.wait()

    return embed


def kernel(x, token_table, pos_table):
    B, S = x.shape
    V, D = token_table.shape
    N = B * S
    embed = _make_embed_kernel(B, S, D, N)
    out = embed(x.astype(jnp.int32), token_table, pos_table)
    return out.reshape(B, S, D)
